# baseline (speedup 1.0000x reference)
import functools

import jax
import jax.numpy as jnp
from jax.experimental import pallas as pl
from jax.experimental.pallas import tpu as pltpu


def _fused_kernel(x_ref, xt_ref, xb_ref, w1_ref, w2_ref, b2_ref, alpha_ref,
                  o_ref, *, cin, cout, rows, w, n_steps):
    rw = rows * w
    s = pl.program_id(1)
    w1 = w1_ref[...].astype(jnp.bfloat16)
    w2 = w2_ref[...].astype(jnp.bfloat16)
    wc = jnp.dot(w2, w1, preferred_element_type=jnp.float32)
    wc = wc.astype(jnp.bfloat16)

    xm = x_ref[0].astype(jnp.bfloat16).reshape(cin, rw)
    top = xt_ref[0, :, 7, :].astype(jnp.bfloat16)
    bot = xb_ref[0, :, 0, :].astype(jnp.bfloat16)
    zb = jnp.bfloat16(0)
    top = jnp.where(s == 0, zb, top)
    bot = jnp.where(s == n_steps - 1, zb, bot)
    slab = jnp.concatenate([top, xm, bot], axis=1)
    ln = rw + 2 * w

    col = jax.lax.broadcasted_iota(jnp.int32, (cin, ln), 1) & (w - 1)
    z1 = jnp.zeros((cin, 1), jnp.bfloat16)
    d0 = jnp.concatenate([z1, slab[:, :-1]], axis=1)
    d0 = jnp.where(col == 0, zb, d0)
    d2 = jnp.concatenate([slab[:, 1:], z1], axis=1)
    d2 = jnp.where(col == w - 1, zb, d2)
    dstack = jnp.concatenate([d0, slab, d2], axis=0)

    b2 = b2_ref[...].astype(jnp.bfloat16)
    alpha = alpha_ref[0].astype(jnp.bfloat16)

    y = None
    for dy in range(3):
        rhs = dstack[:, dy * w: dy * w + rw]
        part = jnp.dot(wc[:, dy * 3 * cin:(dy + 1) * 3 * cin], rhs,
                       preferred_element_type=jnp.float32)
        y = part if y is None else y + part
    y = y.astype(jnp.bfloat16) + b2
    y = jnp.where(y >= 0, y, alpha * y)
    o_ref[0] = y.reshape(cout, rows, w).astype(o_ref.dtype)


def kernel(x_nchw, w1_torch, w2_torch, b2, alpha):
    n, cin, h, w = x_nchw.shape
    ctmp = w1_torch.shape[0]
    cout = w2_torch.shape[0]
    k = w1_torch.shape[2]
    assert k == 3 and w == 128, "kernel specialized to k=3, W=128 lanes"
    kkcin = k * k * cin
    n_steps = 2
    rows = h // n_steps
    g = rows // 8

    w1r = jnp.transpose(w1_torch, (0, 2, 3, 1)).reshape(ctmp, kkcin)
    w2m = w2_torch[:, :, 0, 0]
    b2c = b2.reshape(cout, 1).astype(jnp.float32)
    alpha = alpha.astype(jnp.float32)

    body = functools.partial(_fused_kernel, cin=cin, cout=cout, rows=rows,
                             w=w, n_steps=n_steps)
    out = pl.pallas_call(
        body,
        out_shape=jax.ShapeDtypeStruct((n, cout, h, w), x_nchw.dtype),
        grid_spec=pltpu.PrefetchScalarGridSpec(
            num_scalar_prefetch=0,
            grid=(n, n_steps),
            in_specs=[
                pl.BlockSpec((1, cin, rows, w), lambda i, s: (i, 0, s, 0)),
                pl.BlockSpec((1, cin, 8, w),
                             lambda i, s: (i, 0, jnp.maximum(s * g - 1, 0), 0)),
                pl.BlockSpec((1, cin, 8, w),
                             lambda i, s, _m=h // 8 - 1:
                                 (i, 0, jnp.minimum(s * g + g, _m), 0)),
                pl.BlockSpec((ctmp, kkcin), lambda i, s: (0, 0)),
                pl.BlockSpec((cout, ctmp), lambda i, s: (0, 0)),
                pl.BlockSpec((cout, 1), lambda i, s: (0, 0)),
                pl.BlockSpec(memory_space=pltpu.MemorySpace.SMEM),
            ],
            out_specs=pl.BlockSpec((1, cout, rows, w),
                                   lambda i, s: (i, 0, s, 0)),
        ),
        compiler_params=pltpu.CompilerParams(
            dimension_semantics=("parallel", "arbitrary")),
    )(x_nchw, x_nchw, x_nchw, w1r, w2m, b2c, alpha)
    return out


# speedup vs baseline: 1.0920x; 1.0920x over previous
import functools

import jax
import jax.numpy as jnp
from jax.experimental import pallas as pl
from jax.experimental.pallas import tpu as pltpu


def _fused_kernel(x_ref, wp_ref, o_ref, *, cin, cout, ctmp, h, w, lane_tile):
    hw = h * w
    w1 = wp_ref[0:ctmp, :].astype(jnp.bfloat16)
    w2 = wp_ref[ctmp:ctmp + cout, 0:ctmp].astype(jnp.bfloat16)
    b2 = wp_ref[ctmp:ctmp + cout, ctmp:ctmp + 1].astype(jnp.bfloat16)
    alpha = wp_ref[ctmp:ctmp + 1, ctmp + 1:ctmp + 2][0, 0].astype(jnp.bfloat16)
    wc = jnp.dot(w2, w1, preferred_element_type=jnp.float32)
    wc = wc.astype(jnp.bfloat16)

    xb = x_ref[0].astype(jnp.bfloat16).reshape(cin, hw)
    zrow = jnp.zeros((cin, w), jnp.bfloat16)
    xp = jnp.concatenate([zrow, xb, zrow], axis=1)
    ln = hw + 2 * w

    col = jax.lax.broadcasted_iota(jnp.int32, (cin, ln), 1) & (w - 1)
    z1 = jnp.zeros((cin, 1), jnp.bfloat16)
    zb = jnp.bfloat16(0)
    d0 = jnp.concatenate([z1, xp[:, :-1]], axis=1)
    d0 = jnp.where(col == 0, zb, d0)
    d2 = jnp.concatenate([xp[:, 1:], z1], axis=1)
    d2 = jnp.where(col == w - 1, zb, d2)
    dstack = jnp.concatenate([d0, xp, d2], axis=0)

    for p0 in range(0, hw, lane_tile):
        y = None
        for dy in range(3):
            rhs = dstack[:, p0 + dy * w: p0 + dy * w + lane_tile]
            part = jnp.dot(wc[:, dy * 3 * cin:(dy + 1) * 3 * cin], rhs,
                           preferred_element_type=jnp.float32)
            y = part if y is None else y + part
        y = y.astype(jnp.bfloat16) + b2
        y = jnp.where(y >= 0, y, alpha * y)
        y = y.reshape(cout, lane_tile // w, w).astype(o_ref.dtype)
        o_ref[0, :, p0 // w:(p0 + lane_tile) // w, :] = y


def kernel(x_nchw, w1_torch, w2_torch, b2, alpha):
    n, cin, h, w = x_nchw.shape
    ctmp = w1_torch.shape[0]
    cout = w2_torch.shape[0]
    k = w1_torch.shape[2]
    assert k == 3 and w == 128, "kernel specialized to k=3, W=128 lanes"
    hw = h * w
    kkcin = k * k * cin
    lane_tile = min(8192, hw)
    assert hw % lane_tile == 0

    w1r = jnp.transpose(w1_torch, (0, 2, 3, 1)).reshape(ctmp, kkcin)
    w2m = w2_torch[:, :, 0, 0]
    b2c = b2.reshape(cout, 1).astype(jnp.float32)
    alpha_col = jnp.broadcast_to(alpha.astype(jnp.float32).reshape(1, 1),
                                 (cout, 1))
    wtail = jnp.concatenate([w2m, b2c, alpha_col], axis=1)
    wtail = jnp.pad(wtail, ((0, 0), (0, kkcin - ctmp - 2)))
    wpack = jnp.concatenate([w1r, wtail], axis=0)

    body = functools.partial(_fused_kernel, cin=cin, cout=cout, ctmp=ctmp,
                             h=h, w=w, lane_tile=lane_tile)
    out = pl.pallas_call(
        body,
        out_shape=jax.ShapeDtypeStruct((n, cout, h, w), x_nchw.dtype),
        grid_spec=pltpu.PrefetchScalarGridSpec(
            num_scalar_prefetch=0,
            grid=(n,),
            in_specs=[
                pl.BlockSpec((1, cin, h, w), lambda i: (i, 0, 0, 0)),
                pl.BlockSpec((ctmp + cout, kkcin), lambda i: (0, 0)),
            ],
            out_specs=pl.BlockSpec((1, cout, h, w), lambda i: (i, 0, 0, 0)),
        ),
        compiler_params=pltpu.CompilerParams(
            dimension_semantics=("parallel",),
            allow_input_fusion=[True, True]),
    )(x_nchw, wpack)
    return out


# speedup vs baseline: 1.1628x; 1.0648x over previous
import functools

import jax
import jax.numpy as jnp
from jax.experimental import pallas as pl
from jax.experimental.pallas import tpu as pltpu


def _fused_kernel(x_ref, wp_ref, o_ref, *, cin, cout, ctmp, h, w, lane_tile):
    hw = h * w
    w1 = wp_ref[0:ctmp, :].astype(jnp.bfloat16)
    w2 = wp_ref[ctmp:ctmp + cout, 0:ctmp].astype(jnp.bfloat16)
    b2 = wp_ref[ctmp:ctmp + cout, ctmp:ctmp + 1].astype(jnp.bfloat16)
    alpha = wp_ref[ctmp:ctmp + 1, ctmp + 1:ctmp + 2][0, 0].astype(jnp.bfloat16)
    wc = jnp.dot(w2, w1, preferred_element_type=jnp.float32)
    wc = wc.astype(jnp.bfloat16)

    xb = x_ref[0].astype(jnp.bfloat16).reshape(cin, hw)
    zrow = jnp.zeros((cin, w), jnp.bfloat16)
    xp = jnp.concatenate([zrow, xb, zrow], axis=1)
    ln = hw + 2 * w

    col = jax.lax.broadcasted_iota(jnp.int32, (cin, ln), 1) & (w - 1)
    z1 = jnp.zeros((cin, 1), jnp.bfloat16)
    zb = jnp.bfloat16(0)
    d0 = jnp.concatenate([z1, xp[:, :-1]], axis=1)
    d0 = jnp.where(col == 0, zb, d0)
    d2 = jnp.concatenate([xp[:, 1:], z1], axis=1)
    d2 = jnp.where(col == w - 1, zb, d2)
    dstack = jnp.concatenate([d0, xp, d2], axis=0)

    for p0 in range(0, hw, lane_tile):
        y = None
        for dy in range(3):
            rhs = dstack[:, p0 + dy * w: p0 + dy * w + lane_tile]
            part = jnp.dot(wc[:, dy * 3 * cin:(dy + 1) * 3 * cin], rhs,
                           preferred_element_type=jnp.float32)
            y = part if y is None else y + part
        y = y.astype(jnp.bfloat16) + b2
        y = jnp.where(y >= 0, y, alpha * y)
        y = y.reshape(cout, lane_tile // w, w).astype(o_ref.dtype)
        o_ref[0, :, p0 // w:(p0 + lane_tile) // w, :] = y


def kernel(x_nchw, w1_torch, w2_torch, b2, alpha):
    n, cin, h, w = x_nchw.shape
    ctmp = w1_torch.shape[0]
    cout = w2_torch.shape[0]
    k = w1_torch.shape[2]
    assert k == 3 and w == 128, "kernel specialized to k=3, W=128 lanes"
    hw = h * w
    kkcin = k * k * cin
    lane_tile = min(16384, hw)
    assert hw % lane_tile == 0

    w1r = jnp.transpose(w1_torch, (0, 2, 3, 1)).reshape(ctmp, kkcin)
    w2m = w2_torch[:, :, 0, 0]
    b2c = b2.reshape(cout, 1).astype(jnp.float32)
    alpha_col = jnp.broadcast_to(alpha.astype(jnp.float32).reshape(1, 1),
                                 (cout, 1))
    wtail = jnp.concatenate([w2m, b2c, alpha_col], axis=1)
    wtail = jnp.pad(wtail, ((0, 0), (0, kkcin - ctmp - 2)))
    wpack = jnp.concatenate([w1r, wtail], axis=0)

    body = functools.partial(_fused_kernel, cin=cin, cout=cout, ctmp=ctmp,
                             h=h, w=w, lane_tile=lane_tile)
    out = pl.pallas_call(
        body,
        out_shape=jax.ShapeDtypeStruct((n, cout, h, w), x_nchw.dtype),
        grid_spec=pltpu.PrefetchScalarGridSpec(
            num_scalar_prefetch=0,
            grid=(n,),
            in_specs=[
                pl.BlockSpec((1, cin, h, w), lambda i: (i, 0, 0, 0)),
                pl.BlockSpec((ctmp + cout, kkcin), lambda i: (0, 0)),
            ],
            out_specs=pl.BlockSpec((1, cout, h, w), lambda i: (i, 0, 0, 0)),
        ),
        compiler_params=pltpu.CompilerParams(
            dimension_semantics=("parallel",),
            allow_input_fusion=[True, True]),
    )(x_nchw, wpack)
    return out
